# Initial kernel scaffold; baseline (speedup 1.0000x reference)
import jax
import jax.numpy as jnp
from jax import lax
from jax.experimental import pallas as pl
from jax.experimental.pallas import tpu as pltpu
from jax.experimental.pallas import tpu_sc as plsc

B = 8
N = 384 * 384
HALF = N // 2
CHUNK = 9216
NCH_HALF = HALF // CHUNK
NCH_FULL = N // CHUNK
LANES = 16
VEC_PER_CHUNK = CHUNK // LANES
INF_BITS = 0x7F800000


def _sc_body(rg, ag, sr, sa, cf, out, gbuf, sbuf, cbuf, acc_v, part_v, out_v,
             shared):
    c = lax.axis_index("c")
    s = lax.axis_index("s")
    image = s // 2
    role = s % 2
    wid = c * 16 + s

    zero = jnp.zeros((LANES,), jnp.float32)
    one = jnp.ones((LANES,), jnp.float32)

    def chunk_stats(g_hbm, s_hbm, base, carry):
        pltpu.sync_copy(g_hbm.at[pl.ds(base, CHUNK)], gbuf)
        pltpu.sync_copy(s_hbm.at[pl.ds(base, CHUNK)], sbuf)
        pltpu.sync_copy(cf.at[pl.ds(base, CHUNK)], cbuf)

        def inner(j, carry):
            tot, pos, cnt = carry
            off = j * LANES
            g = gbuf[pl.ds(off, LANES)]
            sc = sbuf[pl.ds(off, LANES)]
            cc = cbuf[pl.ds(off, LANES)]
            d = sc - g
            pre = d * d * cc
            m = g >= 0.1
            return (tot + pre,
                    pos + jnp.where(m, pre, zero),
                    cnt + jnp.where(m, one, zero))

        return lax.fori_loop(0, VEC_PER_CHUNK, inner, carry)

    def phase1(g_hbm, s_hbm):
        base0 = image * N + role * HALF

        def outer(i, carry):
            return chunk_stats(g_hbm, s_hbm, base0 + i * CHUNK, carry)

        tot, pos, cnt = lax.fori_loop(0, NCH_HALF, outer, (zero, zero, zero))
        acc_v[0] = tot
        acc_v[1] = pos
        acc_v[2] = cnt

    pl.when(c == 0)(lambda: phase1(rg, sr))
    pl.when(c == 1)(lambda: phase1(ag, sa))

    pltpu.sync_copy(acc_v, shared.at[s])
    plsc.subcore_barrier()
    partner = s ^ 1
    pltpu.sync_copy(shared.at[partner], part_v)

    tot_sum = jnp.sum(acc_v[0] + part_v[0])
    pos_sum = jnp.sum(acc_v[1] + part_v[1])
    p_f = jnp.sum(acc_v[2] + part_v[2])
    n_f = jnp.float32(N) - p_f
    k_f = 3.0 * p_f

    need_p0 = p_f == 0.0
    need_topk = jnp.logical_and(p_f > 0.0, n_f >= k_f)
    rare = jnp.logical_or(need_p0, need_topk)

    contrib = pos_sum / p_f + (tot_sum - pos_sum) / n_f
    idx = lax.iota(jnp.int32, LANES)
    lane0 = idx == 0
    is_owner = role == 0
    out_v[...] = jnp.where(jnp.logical_and(lane0, is_owner), contrib, zero)

    def count_pass(g_hbm, s_hbm, t_vec, use_all, strict):
        def outer(i, carry):
            base = image * N + i * CHUNK
            pltpu.sync_copy(g_hbm.at[pl.ds(base, CHUNK)], gbuf)
            pltpu.sync_copy(s_hbm.at[pl.ds(base, CHUNK)], sbuf)
            pltpu.sync_copy(cf.at[pl.ds(base, CHUNK)], cbuf)

            def inner(j, carry):
                cntv, sumv = carry
                off = j * LANES
                g = gbuf[pl.ds(off, LANES)]
                sc = sbuf[pl.ds(off, LANES)]
                cc = cbuf[pl.ds(off, LANES)]
                d = sc - g
                pre = d * d * cc
                m = jnp.logical_or(g < 0.1, use_all)
                hit = jnp.logical_and(m, jnp.where(strict, pre > t_vec,
                                                   pre >= t_vec))
                return (cntv + jnp.where(hit, one, zero),
                        sumv + jnp.where(hit, pre, zero))

            return lax.fori_loop(0, VEC_PER_CHUNK, inner, carry)

        cntv, sumv = lax.fori_loop(0, NCH_FULL, outer, (zero, zero))
        return jnp.sum(cntv), jnp.sum(sumv)

    def rare_search(g_hbm, s_hbm):
        k_eff = jnp.where(need_p0, jnp.float32(500.0), k_f)

        def step(_, lohi):
            lo, hi = lohi
            mid = lo + lax.shift_right_logical(hi - lo, 1)
            t_vec = plsc.bitcast(jnp.full((LANES,), mid, jnp.int32),
                                 jnp.float32)
            cnt, _ = count_pass(g_hbm, s_hbm, t_vec, need_p0,
                                jnp.bool_(False))
            ge = cnt >= k_eff
            return (jnp.where(ge, mid, lo), jnp.where(ge, hi, mid))

        lo, _ = lax.fori_loop(0, 31, step,
                              (jnp.int32(0), jnp.int32(INF_BITS)))
        t_vec = plsc.bitcast(jnp.full((LANES,), lo, jnp.int32), jnp.float32)
        t_star = jnp.max(t_vec)
        cnt_gt, sum_gt = count_pass(g_hbm, s_hbm, t_vec, need_p0,
                                    jnp.bool_(True))
        topk_sum = sum_gt + (k_eff - cnt_gt) * t_star
        r_contrib = jnp.where(need_p0, topk_sum / k_eff,
                              pos_sum / p_f + topk_sum / k_eff)
        out_v[...] = jnp.where(lane0, r_contrib, zero)

    do_rare = jnp.logical_and(rare, is_owner)
    pl.when(jnp.logical_and(do_rare, c == 0))(lambda: rare_search(rg, sr))
    pl.when(jnp.logical_and(do_rare, c == 1))(lambda: rare_search(ag, sa))

    pltpu.sync_copy(out_v, out.at[wid])


@jax.jit
def kernel(region_score_GT, affinity_score_GT, score_region, score_affinity,
           confidence):
    rg = region_score_GT.reshape(-1)
    ag = affinity_score_GT.reshape(-1)
    sr = score_region.reshape(-1)
    sa = score_affinity.reshape(-1)
    cf = confidence.reshape(-1)

    mesh = plsc.VectorSubcoreMesh(core_axis_name="c", subcore_axis_name="s")
    out = pl.kernel(
        _sc_body,
        out_type=jax.ShapeDtypeStruct((32, LANES), jnp.float32),
        mesh=mesh,
        scratch_types=[
            pltpu.VMEM((CHUNK,), jnp.float32),
            pltpu.VMEM((CHUNK,), jnp.float32),
            pltpu.VMEM((CHUNK,), jnp.float32),
            pltpu.VMEM((3, LANES), jnp.float32),
            pltpu.VMEM((3, LANES), jnp.float32),
            pltpu.VMEM((LANES,), jnp.float32),
            pltpu.VMEM_SHARED((16, 3, LANES), jnp.float32),
        ],
    )(rg, ag, sr, sa, cf)
    return jnp.sum(out) / B


# speedup vs baseline: 84.4354x; 84.4354x over previous
import jax
import jax.numpy as jnp
from jax import lax
from jax.experimental import pallas as pl
from jax.experimental.pallas import tpu as pltpu
from jax.experimental.pallas import tpu_sc as plsc

B = 8
N = 384 * 384
HALF = N // 2
CHUNK = 9216
NCH_HALF = HALF // CHUNK
NCH_FULL = N // CHUNK
LANES = 16
VEC_PER_CHUNK = CHUNK // LANES
INF_BITS = 0x7F800000


def _sc_body(rg, ag, sr, sa, cf, out, xch, gbuf, sbuf, cbuf, acc_v, part_v,
             out_v):
    c = lax.axis_index("c")
    s = lax.axis_index("s")
    image = lax.shift_right_logical(s, 1)
    role = lax.bitwise_and(s, 1)
    wid = c * 16 + s

    zero = jnp.zeros((LANES,), jnp.float32)
    one = jnp.ones((LANES,), jnp.float32)

    def chunk_stats(g_hbm, s_hbm, base, carry):
        pltpu.sync_copy(g_hbm.at[pl.ds(base, CHUNK)], gbuf)
        pltpu.sync_copy(s_hbm.at[pl.ds(base, CHUNK)], sbuf)
        pltpu.sync_copy(cf.at[pl.ds(base, CHUNK)], cbuf)

        def inner(j, carry):
            tot, pos, cnt = carry
            off = j * LANES
            g = gbuf[pl.ds(off, LANES)]
            sc = sbuf[pl.ds(off, LANES)]
            cc = cbuf[pl.ds(off, LANES)]
            d = sc - g
            pre = d * d * cc
            m = g >= 0.1
            return (tot + pre,
                    pos + jnp.where(m, pre, zero),
                    cnt + jnp.where(m, one, zero))

        return lax.fori_loop(0, VEC_PER_CHUNK, inner, carry)

    def phase1(g_hbm, s_hbm):
        base0 = image * N + role * HALF

        def outer(i, carry):
            return chunk_stats(g_hbm, s_hbm, base0 + i * CHUNK, carry)

        tot, pos, cnt = lax.fori_loop(0, NCH_HALF, outer, (zero, zero, zero))
        acc_v[0] = tot
        acc_v[1] = pos
        acc_v[2] = cnt

    pl.when(c == 0)(lambda: phase1(rg, sr))
    pl.when(c == 1)(lambda: phase1(ag, sa))

    pltpu.sync_copy(acc_v, xch.at[wid])
    plsc.subcore_barrier()
    partner = s ^ 1
    pltpu.sync_copy(xch.at[c * 16 + partner], part_v)

    tot_sum = jnp.sum(acc_v[0] + part_v[0])
    pos_sum = jnp.sum(acc_v[1] + part_v[1])
    p_f = jnp.sum(acc_v[2] + part_v[2])
    n_f = jnp.float32(N) - p_f
    k_f = 3.0 * p_f

    need_p0 = p_f == 0.0
    need_topk = jnp.logical_and(p_f > 0.0, n_f >= k_f)
    rare = jnp.logical_or(need_p0, need_topk)

    def bcast(x):
        return jnp.full((LANES,), x, jnp.float32)

    contrib_v = bcast(pos_sum) / bcast(p_f) + bcast(tot_sum - pos_sum) / bcast(n_f)
    idx = lax.iota(jnp.int32, LANES)
    lane0 = idx == 0
    is_owner = role == 0
    out_v[...] = jnp.where(jnp.logical_and(lane0, is_owner), contrib_v, zero)

    def count_pass(g_hbm, s_hbm, t_vec, use_all, strict):
        def outer(i, carry):
            base = image * N + i * CHUNK
            pltpu.sync_copy(g_hbm.at[pl.ds(base, CHUNK)], gbuf)
            pltpu.sync_copy(s_hbm.at[pl.ds(base, CHUNK)], sbuf)
            pltpu.sync_copy(cf.at[pl.ds(base, CHUNK)], cbuf)

            def inner(j, carry):
                cntv, sumv = carry
                off = j * LANES
                g = gbuf[pl.ds(off, LANES)]
                sc = sbuf[pl.ds(off, LANES)]
                cc = cbuf[pl.ds(off, LANES)]
                d = sc - g
                pre = d * d * cc
                m = jnp.logical_or(g < 0.1, use_all)
                hit = jnp.logical_and(m, jnp.where(strict, pre > t_vec,
                                                   pre >= t_vec))
                return (cntv + jnp.where(hit, one, zero),
                        sumv + jnp.where(hit, pre, zero))

            return lax.fori_loop(0, VEC_PER_CHUNK, inner, carry)

        cntv, sumv = lax.fori_loop(0, NCH_FULL, outer, (zero, zero))
        return jnp.sum(cntv), jnp.sum(sumv)

    def rare_search(g_hbm, s_hbm):
        k_eff = jnp.where(need_p0, jnp.float32(500.0), k_f)

        def step(_, lohi):
            lo, hi = lohi
            mid = lo + lax.shift_right_logical(hi - lo, 1)
            t_vec = plsc.bitcast(jnp.full((LANES,), mid, jnp.int32),
                                 jnp.float32)
            cnt, _ = count_pass(g_hbm, s_hbm, t_vec, need_p0,
                                jnp.bool_(False))
            ge = cnt >= k_eff
            return (jnp.where(ge, mid, lo), jnp.where(ge, hi, mid))

        lo, _ = lax.fori_loop(0, 31, step,
                              (jnp.int32(0), jnp.int32(INF_BITS)))
        t_vec = plsc.bitcast(jnp.full((LANES,), lo, jnp.int32), jnp.float32)
        t_star = jnp.max(t_vec)
        cnt_gt, sum_gt = count_pass(g_hbm, s_hbm, t_vec, need_p0,
                                    jnp.bool_(True))
        topk_sum = sum_gt + (k_eff - cnt_gt) * t_star
        topk_term = bcast(topk_sum) / bcast(k_eff)
        pos_term = bcast(pos_sum) / bcast(p_f)
        r_contrib = jnp.where(need_p0, topk_term, pos_term + topk_term)
        out_v[...] = jnp.where(lane0, r_contrib, zero)

    do_rare = jnp.logical_and(rare, is_owner)
    pl.when(jnp.logical_and(do_rare, c == 0))(lambda: rare_search(rg, sr))
    pl.when(jnp.logical_and(do_rare, c == 1))(lambda: rare_search(ag, sa))

    pltpu.sync_copy(out_v, out.at[wid])


@jax.jit
def kernel(region_score_GT, affinity_score_GT, score_region, score_affinity,
           confidence):
    rg = region_score_GT.reshape(-1)
    ag = affinity_score_GT.reshape(-1)
    sr = score_region.reshape(-1)
    sa = score_affinity.reshape(-1)
    cf = confidence.reshape(-1)

    mesh = plsc.VectorSubcoreMesh(core_axis_name="c", subcore_axis_name="s")
    out, _ = pl.kernel(
        _sc_body,
        out_type=(jax.ShapeDtypeStruct((32, LANES), jnp.float32),
                  jax.ShapeDtypeStruct((32, 3, LANES), jnp.float32)),
        mesh=mesh,
        compiler_params=pltpu.CompilerParams(needs_layout_passes=False),
        scratch_types=[
            pltpu.VMEM((CHUNK,), jnp.float32),
            pltpu.VMEM((CHUNK,), jnp.float32),
            pltpu.VMEM((CHUNK,), jnp.float32),
            pltpu.VMEM((3, LANES), jnp.float32),
            pltpu.VMEM((3, LANES), jnp.float32),
            pltpu.VMEM((LANES,), jnp.float32),
        ],
    )(rg, ag, sr, sa, cf)
    return jnp.sum(out) / B


# speedup vs baseline: 124.9036x; 1.4793x over previous
import jax
import jax.numpy as jnp
from jax import lax
from jax.experimental import pallas as pl
from jax.experimental.pallas import tpu as pltpu
from jax.experimental.pallas import tpu_sc as plsc

B = 8
N = 384 * 384
HALF = N // 2
CHUNK = 18432
NCH_HALF = HALF // CHUNK
NCH_FULL = N // CHUNK
LANES = 16
VEC_PER_CHUNK = CHUNK // LANES
INF_BITS = 0x7F800000


def _sc_body(rg, ag, sr, sa, cf, out, xch, gbuf, sbuf, cbuf, gbuf1, sbuf1,
             cbuf1, acc_v, part_v, out_v, sem0, sem1):
    c = lax.axis_index("c")
    s = lax.axis_index("s")
    image = lax.shift_right_logical(s, 1)
    role = lax.bitwise_and(s, 1)
    wid = c * 16 + s

    zero = jnp.zeros((LANES,), jnp.float32)
    one = jnp.ones((LANES,), jnp.float32)

    bufs = ((gbuf, sbuf, cbuf, sem0), (gbuf1, sbuf1, cbuf1, sem1))

    def compute_chunk(slot, carry):
        g_b, s_b, c_b, _ = bufs[slot]

        def inner(j, carry):
            tot, pos, cnt = carry
            for u in range(4):
                off = j * (4 * LANES) + u * LANES
                g = g_b[pl.ds(off, LANES)]
                sc = s_b[pl.ds(off, LANES)]
                cc = c_b[pl.ds(off, LANES)]
                d = sc - g
                pre = d * d * cc
                m = g >= 0.1
                tot = tot + pre
                pos = pos + jnp.where(m, pre, zero)
                cnt = cnt + jnp.where(m, one, zero)
            return (tot, pos, cnt)

        return lax.fori_loop(0, VEC_PER_CHUNK // 4, inner, carry)

    def phase1(g_hbm, s_hbm):
        base0 = image * N + role * HALF
        handles = [None, None]

        def start(i, slot):
            base = base0 + i * CHUNK
            g_b, s_b, c_b, sem = bufs[slot]
            handles[slot] = [
                pltpu.async_copy(g_hbm.at[pl.ds(base, CHUNK)], g_b, sem),
                pltpu.async_copy(s_hbm.at[pl.ds(base, CHUNK)], s_b, sem),
                pltpu.async_copy(cf.at[pl.ds(base, CHUNK)], c_b, sem),
            ]

        start(0, 0)
        carry = (zero, zero, zero)
        for i in range(NCH_HALF):
            slot = i % 2
            if i + 1 < NCH_HALF:
                start(i + 1, (i + 1) % 2)
            for h in handles[slot]:
                h.wait()
            carry = compute_chunk(slot, carry)
        tot, pos, cnt = carry
        acc_v[0] = tot
        acc_v[1] = pos
        acc_v[2] = cnt

    pl.when(c == 0)(lambda: phase1(rg, sr))
    pl.when(c == 1)(lambda: phase1(ag, sa))

    pltpu.sync_copy(acc_v, xch.at[wid])
    plsc.subcore_barrier()
    partner = s ^ 1
    pltpu.sync_copy(xch.at[c * 16 + partner], part_v)

    tot_sum = jnp.sum(acc_v[0] + part_v[0])
    pos_sum = jnp.sum(acc_v[1] + part_v[1])
    p_f = jnp.sum(acc_v[2] + part_v[2])
    n_f = jnp.float32(N) - p_f
    k_f = 3.0 * p_f

    need_p0 = p_f == 0.0
    need_topk = jnp.logical_and(p_f > 0.0, n_f >= k_f)
    rare = jnp.logical_or(need_p0, need_topk)

    def bcast(x):
        return jnp.full((LANES,), x, jnp.float32)

    contrib_v = bcast(pos_sum) / bcast(p_f) + bcast(tot_sum - pos_sum) / bcast(n_f)
    idx = lax.iota(jnp.int32, LANES)
    lane0 = idx == 0
    is_owner = role == 0
    out_v[...] = jnp.where(jnp.logical_and(lane0, is_owner), contrib_v, zero)

    def count_pass(g_hbm, s_hbm, t_vec, use_all, strict):
        def outer(i, carry):
            base = image * N + i * CHUNK
            pltpu.sync_copy(g_hbm.at[pl.ds(base, CHUNK)], gbuf)
            pltpu.sync_copy(s_hbm.at[pl.ds(base, CHUNK)], sbuf)
            pltpu.sync_copy(cf.at[pl.ds(base, CHUNK)], cbuf)

            def inner(j, carry):
                cntv, sumv = carry
                off = j * LANES
                g = gbuf[pl.ds(off, LANES)]
                sc = sbuf[pl.ds(off, LANES)]
                cc = cbuf[pl.ds(off, LANES)]
                d = sc - g
                pre = d * d * cc
                m = jnp.logical_or(g < 0.1, use_all)
                hit = jnp.logical_and(m, jnp.where(strict, pre > t_vec,
                                                   pre >= t_vec))
                return (cntv + jnp.where(hit, one, zero),
                        sumv + jnp.where(hit, pre, zero))

            return lax.fori_loop(0, VEC_PER_CHUNK, inner, carry)

        cntv, sumv = lax.fori_loop(0, NCH_FULL, outer, (zero, zero))
        return jnp.sum(cntv), jnp.sum(sumv)

    def rare_search(g_hbm, s_hbm):
        k_eff = jnp.where(need_p0, jnp.float32(500.0), k_f)

        def step(_, lohi):
            lo, hi = lohi
            mid = lo + lax.shift_right_logical(hi - lo, 1)
            t_vec = plsc.bitcast(jnp.full((LANES,), mid, jnp.int32),
                                 jnp.float32)
            cnt, _ = count_pass(g_hbm, s_hbm, t_vec, need_p0,
                                jnp.bool_(False))
            ge = cnt >= k_eff
            return (jnp.where(ge, mid, lo), jnp.where(ge, hi, mid))

        lo, _ = lax.fori_loop(0, 31, step,
                              (jnp.int32(0), jnp.int32(INF_BITS)))
        t_vec = plsc.bitcast(jnp.full((LANES,), lo, jnp.int32), jnp.float32)
        t_star = jnp.max(t_vec)
        cnt_gt, sum_gt = count_pass(g_hbm, s_hbm, t_vec, need_p0,
                                    jnp.bool_(True))
        topk_sum = sum_gt + (k_eff - cnt_gt) * t_star
        topk_term = bcast(topk_sum) / bcast(k_eff)
        pos_term = bcast(pos_sum) / bcast(p_f)
        r_contrib = jnp.where(need_p0, topk_term, pos_term + topk_term)
        out_v[...] = jnp.where(lane0, r_contrib, zero)

    do_rare = jnp.logical_and(rare, is_owner)
    pl.when(jnp.logical_and(do_rare, c == 0))(lambda: rare_search(rg, sr))
    pl.when(jnp.logical_and(do_rare, c == 1))(lambda: rare_search(ag, sa))

    pltpu.sync_copy(out_v, out.at[wid])


@jax.jit
def kernel(region_score_GT, affinity_score_GT, score_region, score_affinity,
           confidence):
    rg = region_score_GT.reshape(-1)
    ag = affinity_score_GT.reshape(-1)
    sr = score_region.reshape(-1)
    sa = score_affinity.reshape(-1)
    cf = confidence.reshape(-1)

    mesh = plsc.VectorSubcoreMesh(core_axis_name="c", subcore_axis_name="s")
    out, _ = pl.kernel(
        _sc_body,
        out_type=(jax.ShapeDtypeStruct((32, LANES), jnp.float32),
                  jax.ShapeDtypeStruct((32, 3, LANES), jnp.float32)),
        mesh=mesh,
        compiler_params=pltpu.CompilerParams(needs_layout_passes=False),
        scratch_types=[
            pltpu.VMEM((CHUNK,), jnp.float32),
            pltpu.VMEM((CHUNK,), jnp.float32),
            pltpu.VMEM((CHUNK,), jnp.float32),
            pltpu.VMEM((CHUNK,), jnp.float32),
            pltpu.VMEM((CHUNK,), jnp.float32),
            pltpu.VMEM((CHUNK,), jnp.float32),
            pltpu.VMEM((3, LANES), jnp.float32),
            pltpu.VMEM((3, LANES), jnp.float32),
            pltpu.VMEM((LANES,), jnp.float32),
            pltpu.SemaphoreType.DMA,
            pltpu.SemaphoreType.DMA,
        ],
    )(rg, ag, sr, sa, cf)
    return jnp.sum(out) / B
